# baseline (speedup 1.0000x reference)
import functools

import jax
import jax.numpy as jnp
from jax.experimental import pallas as pl
from jax.experimental.pallas import tpu as pltpu

_TB = 64
_LANES = 128
_NEG = -1e30


def _logaddexp(a, b):
    m = jnp.maximum(a, b)
    return m + jnp.log1p(jnp.exp(-jnp.abs(a - b)))


def _reduce_kernel(logits_ref, lab_ref, emit_ref, blank_ref):
    x = logits_ref[0]
    n_t, up1, v = x.shape
    labm = lab_ref[0]
    vio = jax.lax.broadcasted_iota(jnp.int32, (up1, v), 1)
    maskf = jnp.where(vio == labm, 1.0, 0.0)
    mask0 = jnp.where(vio == 0, 1.0, 0.0)
    emit = jnp.sum(x * maskf[None], axis=-1)
    blank = jnp.sum(x * mask0[None], axis=-1)
    pad = jnp.zeros((n_t, _LANES - up1), jnp.float32)
    emit = jnp.concatenate([emit, pad], axis=-1)
    blank = jnp.concatenate([blank, pad], axis=-1)
    emit_ref[...] = emit[None]
    blank_ref[...] = blank[None]


def _skew(x, rows):
    t = x.shape[0]
    x = jnp.concatenate(
        [x, jnp.full((rows - t, _LANES), _NEG, jnp.float32)], axis=0)
    lane = jax.lax.broadcasted_iota(jnp.int32, (1, _LANES), 1)
    for k in (1, 2, 4, 8, 16, 32, 64):
        shifted = jnp.concatenate(
            [jnp.full((k, _LANES), _NEG, jnp.float32), x[:-k]], axis=0)
        x = jnp.where((lane & k) != 0, shifted, x)
    return x


def _alpha_kernel(emit_ref, blank_ref, tl_ref, ul_ref, out_ref,
                  se_ref, sb_ref, *, n_d):
    rows = se_ref.shape[0]
    nb = se_ref.shape[1]

    se_ref[...] = jnp.concatenate(
        [_skew(emit_ref[q], rows)[:, None] for q in range(nb)], axis=1)
    sb_ref[...] = jnp.concatenate(
        [_skew(blank_ref[q], rows)[:, None] for q in range(nb)], axis=1)

    lane = jax.lax.broadcasted_iota(jnp.int32, (nb, _LANES), 1)
    sub = jax.lax.broadcasted_iota(jnp.int32, (nb, _LANES), 0)
    tls = [tl_ref[q] for q in range(nb)]
    uls = [ul_ref[q] for q in range(nb)]
    tl_v = tls[nb - 1]
    ul_v = uls[nb - 1]
    for q in range(nb - 2, -1, -1):
        tl_v = jnp.where(sub == q, tls[q], tl_v)
        ul_v = jnp.where(sub == q, uls[q], ul_v)
    dstar = tl_v + ul_v
    usel_b = lane == ul_v

    d_init = jnp.where(lane == 0, 0.0, _NEG)
    vacc0 = jnp.zeros((nb, _LANES), jnp.float32)
    negcol = jnp.full((nb, 1), _NEG, jnp.float32)

    def body(d, carry):
        dvec, vacc = carry
        ed = se_ref[pl.ds(d - 1, 1), :, :][0]
        bd = sb_ref[pl.ds(d - 1, 1), :, :][0]
        tmp = dvec + ed
        sh = jnp.concatenate([negcol, tmp[:, :-1]], axis=-1)
        dnew = _logaddexp(dvec + bd, sh)
        vacc = vacc + jnp.where(dstar == d, jnp.where(usel_b, dnew, 0.0), 0.0)
        return dnew, vacc

    d_hi = tls[0] + uls[0]
    for q in range(1, nb):
        d_hi = jnp.maximum(d_hi, tls[q] + uls[q])
    d_hi = jnp.minimum(d_hi, n_d)
    _, vacc = jax.lax.fori_loop(1, d_hi + 1, body, (d_init, vacc0))
    out_ref[0] = vacc


def kernel(logits, labels, logit_lengths, label_lengths):
    B, T, up1, V = logits.shape
    U = up1 - 1
    tl = jnp.clip(logit_lengths, 1, T).astype(jnp.int32) - 1
    ul = jnp.clip(label_lengths, 1, U).astype(jnp.int32)
    labs = jnp.clip(labels, 0, V - 1).astype(jnp.int32)
    lab_col = jnp.concatenate(
        [labs, jnp.full((B, 1), -1, jnp.int32)], axis=1).reshape(B, up1, 1)

    emit, blank = pl.pallas_call(
        _reduce_kernel,
        out_shape=(
            jax.ShapeDtypeStruct((B, T, _LANES), jnp.float32),
            jax.ShapeDtypeStruct((B, T, _LANES), jnp.float32),
        ),
        grid=(2, B // 2, T // _TB),
        in_specs=[
            pl.BlockSpec((1, _TB, up1, V), lambda c, i, t: (c * 2 + i, t, 0, 0)),
            pl.BlockSpec((1, up1, 1), lambda c, i, t: (c * 2 + i, 0, 0)),
        ],
        out_specs=(
            pl.BlockSpec((1, _TB, _LANES), lambda c, i, t: (c * 2 + i, t, 0)),
            pl.BlockSpec((1, _TB, _LANES), lambda c, i, t: (c * 2 + i, t, 0)),
        ),
        compiler_params=pltpu.CompilerParams(
            dimension_semantics=("parallel", "parallel", "parallel"),
        ),
        name="rnnt_reduce",
    )(logits, lab_col)

    out = pl.pallas_call(
        functools.partial(_alpha_kernel, n_d=T - 1 + U),
        out_shape=jax.ShapeDtypeStruct((1, B, _LANES), jnp.float32),
        grid=(1,),
        in_specs=[
            pl.BlockSpec((B, T, _LANES), lambda p: (0, 0, 0)),
            pl.BlockSpec((B, T, _LANES), lambda p: (0, 0, 0)),
            pl.BlockSpec(memory_space=pltpu.SMEM),
            pl.BlockSpec(memory_space=pltpu.SMEM),
        ],
        out_specs=pl.BlockSpec((1, B, _LANES), lambda p: (0, 0, 0)),
        scratch_shapes=[
            pltpu.VMEM((T + _LANES, B, _LANES), jnp.float32),
            pltpu.VMEM((T + _LANES, B, _LANES), jnp.float32),
        ],
        compiler_params=pltpu.CompilerParams(
            dimension_semantics=("arbitrary",),
        ),
        name="rnnt_alpha",
    )(emit, blank, tl, ul)
    return (-jnp.sum(out) / B).reshape(1)


# speedup vs baseline: 1.0034x; 1.0034x over previous
import functools

import jax
import jax.numpy as jnp
from jax.experimental import pallas as pl
from jax.experimental.pallas import tpu as pltpu

_TB = 32
_LANES = 128
_NEG = -1e30


def _logaddexp(a, b):
    m = jnp.maximum(a, b)
    return m + jnp.log1p(jnp.exp(-jnp.abs(a - b)))


def _reduce_kernel(logits_ref, lab_ref, emit_ref, blank_ref):
    x = logits_ref[0]
    n_t, up1, v = x.shape
    labm = lab_ref[0]
    vio = jax.lax.broadcasted_iota(jnp.int32, (up1, v), 1)
    maskf = jnp.where(vio == labm, 1.0, 0.0)
    mask0 = jnp.where(vio == 0, 1.0, 0.0)
    emit = jnp.sum(x * maskf[None], axis=-1)
    blank = jnp.sum(x * mask0[None], axis=-1)
    pad = jnp.zeros((n_t, _LANES - up1), jnp.float32)
    emit = jnp.concatenate([emit, pad], axis=-1)
    blank = jnp.concatenate([blank, pad], axis=-1)
    emit_ref[...] = emit[None]
    blank_ref[...] = blank[None]


def _skew(x, rows):
    t = x.shape[0]
    x = jnp.concatenate(
        [x, jnp.full((rows - t, _LANES), _NEG, jnp.float32)], axis=0)
    lane = jax.lax.broadcasted_iota(jnp.int32, (1, _LANES), 1)
    for k in (1, 2, 4, 8, 16, 32, 64):
        shifted = jnp.concatenate(
            [jnp.full((k, _LANES), _NEG, jnp.float32), x[:-k]], axis=0)
        x = jnp.where((lane & k) != 0, shifted, x)
    return x


def _alpha_kernel(emit_ref, blank_ref, tl_ref, ul_ref, out_ref,
                  se_ref, sb_ref, *, n_d):
    rows = se_ref.shape[0]
    nb = se_ref.shape[1]

    se_ref[...] = jnp.concatenate(
        [_skew(emit_ref[q], rows)[:, None] for q in range(nb)], axis=1)
    sb_ref[...] = jnp.concatenate(
        [_skew(blank_ref[q], rows)[:, None] for q in range(nb)], axis=1)

    lane = jax.lax.broadcasted_iota(jnp.int32, (nb, _LANES), 1)
    sub = jax.lax.broadcasted_iota(jnp.int32, (nb, _LANES), 0)
    tls = [tl_ref[q] for q in range(nb)]
    uls = [ul_ref[q] for q in range(nb)]
    tl_v = tls[nb - 1]
    ul_v = uls[nb - 1]
    for q in range(nb - 2, -1, -1):
        tl_v = jnp.where(sub == q, tls[q], tl_v)
        ul_v = jnp.where(sub == q, uls[q], ul_v)
    dstar = tl_v + ul_v
    usel_b = lane == ul_v

    d_init = jnp.where(lane == 0, 0.0, _NEG)
    vacc0 = jnp.zeros((nb, _LANES), jnp.float32)
    negcol = jnp.full((nb, 1), _NEG, jnp.float32)

    def body(d, carry):
        dvec, vacc = carry
        ed = se_ref[pl.ds(d - 1, 1), :, :][0]
        bd = sb_ref[pl.ds(d - 1, 1), :, :][0]
        tmp = dvec + ed
        sh = jnp.concatenate([negcol, tmp[:, :-1]], axis=-1)
        dnew = _logaddexp(dvec + bd, sh)
        vacc = vacc + jnp.where(dstar == d, jnp.where(usel_b, dnew, 0.0), 0.0)
        return dnew, vacc

    d_hi = tls[0] + uls[0]
    for q in range(1, nb):
        d_hi = jnp.maximum(d_hi, tls[q] + uls[q])
    d_hi = jnp.minimum(d_hi, n_d)
    _, vacc = jax.lax.fori_loop(1, d_hi + 1, body, (d_init, vacc0))
    out_ref[0] = vacc


def kernel(logits, labels, logit_lengths, label_lengths):
    B, T, up1, V = logits.shape
    U = up1 - 1
    tl = jnp.clip(logit_lengths, 1, T).astype(jnp.int32) - 1
    ul = jnp.clip(label_lengths, 1, U).astype(jnp.int32)
    labs = jnp.clip(labels, 0, V - 1).astype(jnp.int32)
    lab_col = jnp.concatenate(
        [labs, jnp.full((B, 1), -1, jnp.int32)], axis=1).reshape(B, up1, 1)

    emit, blank = pl.pallas_call(
        _reduce_kernel,
        out_shape=(
            jax.ShapeDtypeStruct((B, T, _LANES), jnp.float32),
            jax.ShapeDtypeStruct((B, T, _LANES), jnp.float32),
        ),
        grid=(2, B // 2, T // _TB),
        in_specs=[
            pl.BlockSpec((1, _TB, up1, V), lambda c, i, t: (c * 2 + i, t, 0, 0)),
            pl.BlockSpec((1, up1, 1), lambda c, i, t: (c * 2 + i, 0, 0)),
        ],
        out_specs=(
            pl.BlockSpec((1, _TB, _LANES), lambda c, i, t: (c * 2 + i, t, 0)),
            pl.BlockSpec((1, _TB, _LANES), lambda c, i, t: (c * 2 + i, t, 0)),
        ),
        compiler_params=pltpu.CompilerParams(
            dimension_semantics=("parallel", "parallel", "parallel"),
        ),
        name="rnnt_reduce",
    )(logits, lab_col)

    out = pl.pallas_call(
        functools.partial(_alpha_kernel, n_d=T - 1 + U),
        out_shape=jax.ShapeDtypeStruct((1, B, _LANES), jnp.float32),
        grid=(1,),
        in_specs=[
            pl.BlockSpec((B, T, _LANES), lambda p: (0, 0, 0)),
            pl.BlockSpec((B, T, _LANES), lambda p: (0, 0, 0)),
            pl.BlockSpec(memory_space=pltpu.SMEM),
            pl.BlockSpec(memory_space=pltpu.SMEM),
        ],
        out_specs=pl.BlockSpec((1, B, _LANES), lambda p: (0, 0, 0)),
        scratch_shapes=[
            pltpu.VMEM((T + _LANES, B, _LANES), jnp.float32),
            pltpu.VMEM((T + _LANES, B, _LANES), jnp.float32),
        ],
        compiler_params=pltpu.CompilerParams(
            dimension_semantics=("arbitrary",),
        ),
        name="rnnt_alpha",
    )(emit, blank, tl, ul)
    return (-jnp.sum(out) / B).reshape(1)
